# Initial kernel scaffold; baseline (speedup 1.0000x reference)
import jax
import jax.numpy as jnp
from jax.experimental import pallas as pl


def kernel(x, pos_table):
    raise NotImplementedError("write your pallas kernel here")


# speedup vs baseline: 4.7689x; 4.7689x over previous
import jax
import jax.numpy as jnp
from jax.experimental import pallas as pl


_BT = 256


def _body(x_ref, pos_ref, out_ref):
    out_ref[...] = x_ref[...] + pos_ref[...][:, None, :]


def kernel(x, pos_table):
    T, B, D = x.shape
    grid = (T // _BT,)
    return pl.pallas_call(
        _body,
        grid=grid,
        in_specs=[
            pl.BlockSpec((_BT, B, D), lambda i: (i, 0, 0)),
            pl.BlockSpec((_BT, D), lambda i: (i, 0)),
        ],
        out_specs=pl.BlockSpec((_BT, B, D), lambda i: (i, 0, 0)),
        out_shape=jax.ShapeDtypeStruct((T, B, D), x.dtype),
    )(x, pos_table)
